# baseline (speedup 1.0000x reference)
import functools

import jax
import jax.numpy as jnp
from jax import lax
from jax.experimental import pallas as pl
from jax.experimental.pallas import tpu as pltpu
from jax.experimental.pallas import tpu_sc as plsc

D = 128
L = 16
NC = 2
NS = 16
NW = NC * NS

GC = 128
SCC = 40


def _gather_mul_call(x_src, x_dst, src_idx, dst_idx):
    E = src_idx.shape[0]
    per_w = E // NW
    n_chunks = -(-per_w // GC)
    last_base = per_w - GC
    assert n_chunks % 2 == 0

    mesh = plsc.VectorSubcoreMesh(core_axis_name="c", subcore_axis_name="s")

    @functools.partial(
        pl.kernel,
        mesh=mesh,
        out_type=jax.ShapeDtypeStruct((E, D), jnp.float32),
        scratch_types=[
            pltpu.VMEM((per_w,), jnp.int32),
            pltpu.VMEM((per_w,), jnp.int32),
            pltpu.VMEM((GC, D), jnp.float32),
            pltpu.VMEM((GC, D), jnp.float32),
            pltpu.VMEM((GC, D), jnp.float32),
            pltpu.VMEM((GC, D), jnp.float32),
            pltpu.VMEM((GC, D), jnp.float32),
            pltpu.VMEM((GC, D), jnp.float32),
            pltpu.SemaphoreType.DMA,
            pltpu.SemaphoreType.DMA,
            pltpu.SemaphoreType.DMA,
            pltpu.SemaphoreType.DMA,
        ],
    )
    def gather_mul(xs_hbm, xdst_hbm, s_hbm, d_hbm, p_hbm,
                   si_all, di_all, xj0, xj1, xd0, xd1, p0, p1,
                   gs0, gs1, ss0, ss1):
        xj = (xj0, xj1)
        xd = (xd0, xd1)
        pv = (p0, p1)
        gsem = (gs0, gs1)
        ssem = (ss0, ss1)
        wid = lax.axis_index("s") * NC + lax.axis_index("c")
        w_base = wid * per_w

        pltpu.sync_copy(s_hbm.at[pl.ds(w_base, per_w)], si_all)
        pltpu.sync_copy(d_hbm.at[pl.ds(w_base, per_w)], di_all)

        def off(c):
            return pl.multiple_of(jnp.minimum(c * GC, last_base), 8)

        def start_gathers(c, s):
            o = off(c)
            pltpu.async_copy(xs_hbm.at[si_all.at[pl.ds(o, GC)]], xj[s], gsem[s])
            pltpu.async_copy(xdst_hbm.at[di_all.at[pl.ds(o, GC)]], xd[s], gsem[s])

        def wait_gathers(s):
            z = pl.ds(0, GC)
            pltpu.make_async_copy(xs_hbm.at[si_all.at[z]], xj[s], gsem[s]).wait()
            pltpu.make_async_copy(xdst_hbm.at[di_all.at[z]], xd[s], gsem[s]).wait()

        def wait_store(s):
            pltpu.make_async_copy(pv[s], p_hbm.at[pl.ds(0, GC)], ssem[s]).wait()

        start_gathers(0, 0)

        def pair_body(i2, _):
            for s in (0, 1):
                c = 2 * i2 + s

                @pl.when(c + 1 < n_chunks)
                def _():
                    start_gathers(c + 1, 1 - s)

                wait_gathers(s)

                @pl.when(c >= 2)
                def _():
                    wait_store(s)

                @plsc.parallel_loop(0, GC, unroll=8)
                def _(r):
                    for j in range(D // L):
                        sl = pl.ds(j * L, L)
                        pv[s][r, sl] = xj[s][r, sl] * xd[s][r, sl]

                base = pl.multiple_of(w_base + off(c), 8)
                pltpu.async_copy(pv[s], p_hbm.at[pl.ds(base, GC)], ssem[s])
            return 0

        lax.fori_loop(0, n_chunks // 2, pair_body, 0)
        wait_store(0)
        wait_store(1)

    return gather_mul(x_src, x_dst, src_idx, dst_idx)


def _linear_lrelu_call(p, W, b):
    E = p.shape[0]
    BLK = 2000

    def body(p_ref, W_ref, b_ref, o_ref):
        dn = (((1,), (1,)), ((), ()))
        z = lax.dot_general(p_ref[...], W_ref[...], dn,
                            preferred_element_type=jnp.float32) + b_ref[...]
        o_ref[...] = jnp.where(z >= 0, z, 0.01 * z)

    return pl.pallas_call(
        body,
        grid=(E // BLK,),
        in_specs=[
            pl.BlockSpec((BLK, D), lambda i: (i, 0)),
            pl.BlockSpec((D, D), lambda i: (0, 0)),
            pl.BlockSpec((1, D), lambda i: (0, 0)),
        ],
        out_specs=pl.BlockSpec((BLK, D), lambda i: (i, 0)),
        out_shape=jax.ShapeDtypeStruct((E, D), jnp.float32),
    )(p, W, b.reshape(1, D))


def _scatter_call(m, dst2, zeros_nd, n_rows):
    E = m.shape[0]
    per_t = E // NW
    n_chunks = per_t // SCC
    stripe = 624
    stripe_last = n_rows - (NS - 1) * stripe

    mesh = plsc.VectorSubcoreMesh(core_axis_name="c", subcore_axis_name="s")

    @functools.partial(
        pl.kernel,
        mesh=mesh,
        out_type=jax.ShapeDtypeStruct((NC, n_rows, D), jnp.float32),
        scratch_types=[
            pltpu.VMEM((n_chunks, SCC), jnp.int32),
            pltpu.VMEM((SCC, D), jnp.float32),
            pltpu.VMEM((SCC, D), jnp.float32),
            pltpu.VMEM_SHARED((10000, D), jnp.float32),
            pltpu.SemaphoreType.DMA,
            pltpu.SemaphoreType.DMA,
            pltpu.SemaphoreType.DMA,
            pltpu.SemaphoreType.DMA,
        ],
    )
    def scatter(m_hbm, d_hbm, z_hbm, o_hbm, d_all, mrow0, mrow1, acc_sh,
                ls0, ls1, cs0, cs1):
        mrow = (mrow0, mrow1)
        lsem = (ls0, ls1)
        csem = (cs0, cs1)
        cid = lax.axis_index("c")
        tid = lax.axis_index("s")
        wid = tid * NC + cid
        t_base = wid * per_t

        @pl.when(tid < NS - 1)
        def _():
            off = pl.multiple_of(tid * stripe, 8)
            pltpu.sync_copy(z_hbm.at[pl.ds(off, stripe)],
                            acc_sh.at[pl.ds(off, stripe)])

        @pl.when(tid == NS - 1)
        def _():
            off = (NS - 1) * stripe
            pltpu.sync_copy(z_hbm.at[pl.ds(off, stripe_last)],
                            acc_sh.at[pl.ds(off, stripe_last)])

        pltpu.sync_copy(d_hbm.at[wid], d_all)
        plsc.subcore_barrier()

        def start_load(c, s):
            base = pl.multiple_of(t_base + c * SCC, 8)
            pltpu.async_copy(m_hbm.at[pl.ds(base, SCC)], mrow[s], lsem[s])

        def wait_load(s):
            pltpu.make_async_copy(m_hbm.at[pl.ds(t_base, SCC)],
                                  mrow[s], lsem[s]).wait()

        def wait_scat(s):
            pltpu.make_async_copy(mrow[s], acc_sh.at[d_all.at[0]],
                                  csem[s]).wait()

        def do_chunk(c, s, may_be_first):
            wait_load(s)
            pltpu.async_copy(mrow[s], acc_sh.at[d_all.at[c]], csem[s],
                             add=True)
            if may_be_first:
                @pl.when(c >= 1)
                def _():
                    wait_scat(1 - s)
            else:
                wait_scat(1 - s)

            @pl.when(c + 1 < n_chunks)
            def _():
                start_load(c + 1, 1 - s)

        start_load(0, 0)

        def pair_body(i2, _):
            c = 2 * i2
            do_chunk(c, 0, True)
            do_chunk(c + 1, 1, False)
            return 0

        lax.fori_loop(0, n_chunks // 2, pair_body, 0)
        if n_chunks % 2 == 1:
            do_chunk(n_chunks - 1, 0, False)
        wait_scat((n_chunks - 1) % 2)

        plsc.subcore_barrier()

        @pl.when(tid < NS - 1)
        def _():
            off = pl.multiple_of(tid * stripe, 8)
            pltpu.sync_copy(acc_sh.at[pl.ds(off, stripe)],
                            o_hbm.at[cid, pl.ds(off, stripe)])

        @pl.when(tid == NS - 1)
        def _():
            off = (NS - 1) * stripe
            pltpu.sync_copy(acc_sh.at[pl.ds(off, stripe_last)],
                            o_hbm.at[cid, pl.ds(off, stripe_last)])

    return scatter(m, dst2, zeros_nd)


def _ln_relu_call(oi_parts, ou_parts, ln_g_item, ln_b_item, ln_g_user, ln_b_user):
    n = oi_parts.shape[1]
    BLKN = 2000

    def body(x1_ref, x2_ref, g1_ref, b1_ref, g2_ref, b2_ref, o1_ref, o2_ref):
        for x_ref, g_ref, b_ref, o_ref in (
            (x1_ref, g1_ref, b1_ref, o1_ref),
            (x2_ref, g2_ref, b2_ref, o2_ref),
        ):
            x = x_ref[0] + x_ref[1]
            mu = jnp.mean(x, axis=-1, keepdims=True)
            var = jnp.mean((x - mu) ** 2, axis=-1, keepdims=True)
            y = (x - mu) / jnp.sqrt(var + 1e-5) * g_ref[...] + b_ref[...]
            o_ref[...] = jnp.maximum(y, 0.0)

    part_spec = pl.BlockSpec((NC, BLKN, D), lambda i: (0, i, 0))
    vec_spec = pl.BlockSpec((1, D), lambda i: (0, 0))
    return pl.pallas_call(
        body,
        grid=(n // BLKN,),
        in_specs=[part_spec, part_spec, vec_spec, vec_spec, vec_spec, vec_spec],
        out_specs=[
            pl.BlockSpec((BLKN, D), lambda i: (i, 0)),
            pl.BlockSpec((BLKN, D), lambda i: (i, 0)),
        ],
        out_shape=(jax.ShapeDtypeStruct((n, D), jnp.float32),
                   jax.ShapeDtypeStruct((n, D), jnp.float32)),
    )(oi_parts, ou_parts, ln_g_item.reshape(1, D), ln_b_item.reshape(1, D),
      ln_g_user.reshape(1, D), ln_b_user.reshape(1, D))


def kernel(x_user, x_item, edge_index_user_item, edge_index_item_user,
           W_ui, b_ui, W_iu, b_iu,
           ln_g_user, ln_b_user, ln_g_item, ln_b_item):
    n_user = x_user.shape[0]
    n_item = x_item.shape[0]

    src_ui = edge_index_user_item[0].astype(jnp.int32)
    dst_ui = edge_index_user_item[1].astype(jnp.int32)
    src_iu = edge_index_item_user[0].astype(jnp.int32)
    dst_iu = edge_index_item_user[1].astype(jnp.int32)

    zeros_nd = jnp.zeros((n_item, D), jnp.float32)
    dst2_ui = dst_ui.reshape(NW, -1, SCC)
    dst2_iu = dst_iu.reshape(NW, -1, SCC)

    p_ui = _gather_mul_call(x_user, x_item, src_ui, dst_ui)
    m_ui = _linear_lrelu_call(p_ui, W_ui, b_ui)
    p_iu = _gather_mul_call(x_item, x_user, src_iu, dst_iu)
    m_iu = _linear_lrelu_call(p_iu, W_iu, b_iu)

    oi_parts = _scatter_call(m_ui, dst2_ui, zeros_nd, n_item)
    ou_parts = _scatter_call(m_iu, dst2_iu, zeros_nd, n_user)

    out_item, out_user = _ln_relu_call(oi_parts, ou_parts,
                                       ln_g_item, ln_b_item,
                                       ln_g_user, ln_b_user)
    return (out_user, out_item)


# speedup vs baseline: 1.1519x; 1.1519x over previous
import functools

import numpy as np
import jax
import jax.numpy as jnp
from jax import lax
from jax.experimental import pallas as pl
from jax.experimental.pallas import tpu as pltpu
from jax.experimental.pallas import tpu_sc as plsc

D = 128
L = 16
NC = 2
NS = 16
NW = NC * NS

GC = 128
SCC = 128
SCT = 16

def _gather_mul_call(x_user, x_item, src_ui, dst_ui, src_iu, dst_iu):
    E = src_ui.shape[0]
    per_w = E // NW
    n_chunks = -(-per_w // GC)
    last_base = per_w - GC
    assert n_chunks % 2 == 0

    mesh = plsc.VectorSubcoreMesh(core_axis_name="c", subcore_axis_name="s")

    @functools.partial(
        pl.kernel,
        mesh=mesh,
        out_type=(jax.ShapeDtypeStruct((E, D), jnp.float32),
                  jax.ShapeDtypeStruct((E, D), jnp.float32)),
        scratch_types=[
            pltpu.VMEM((per_w,), jnp.int32),
            pltpu.VMEM((per_w,), jnp.int32),
            pltpu.VMEM((GC, D), jnp.float32),
            pltpu.VMEM((GC, D), jnp.float32),
            pltpu.VMEM((GC, D), jnp.float32),
            pltpu.VMEM((GC, D), jnp.float32),
            pltpu.VMEM((GC, D), jnp.float32),
            pltpu.VMEM((GC, D), jnp.float32),
            pltpu.SemaphoreType.DMA,
            pltpu.SemaphoreType.DMA,
            pltpu.SemaphoreType.DMA,
            pltpu.SemaphoreType.DMA,
        ],
    )
    def gather_mul(xu_hbm, xi_hbm, sui_hbm, dui_hbm, siu_hbm, diu_hbm,
                   p_ui_hbm, p_iu_hbm,
                   si_all, di_all, xj0, xj1, xd0, xd1, p0, p1,
                   gs0, gs1, ss0, ss1):
        xj = (xj0, xj1)
        xd = (xd0, xd1)
        pv = (p0, p1)
        gsem = (gs0, gs1)
        ssem = (ss0, ss1)
        wid = lax.axis_index("s") * NC + lax.axis_index("c")
        w_base = wid * per_w

        def one_type(xs_hbm, xdst_hbm, s_hbm, d_hbm, p_hbm, first):
            pltpu.sync_copy(s_hbm.at[pl.ds(w_base, per_w)], si_all)
            pltpu.sync_copy(d_hbm.at[pl.ds(w_base, per_w)], di_all)

            def off(c):
                return pl.multiple_of(jnp.minimum(c * GC, last_base), 8)

            def start_gathers(c, s):
                o = off(c)
                pltpu.async_copy(xs_hbm.at[si_all.at[pl.ds(o, GC)]], xj[s], gsem[s])
                pltpu.async_copy(xdst_hbm.at[di_all.at[pl.ds(o, GC)]], xd[s], gsem[s])

            def wait_gathers(s):
                z = pl.ds(0, GC)
                pltpu.make_async_copy(xs_hbm.at[si_all.at[z]], xj[s], gsem[s]).wait()
                pltpu.make_async_copy(xdst_hbm.at[di_all.at[z]], xd[s], gsem[s]).wait()

            def wait_store(s):
                pltpu.make_async_copy(pv[s], p_hbm.at[pl.ds(0, GC)], ssem[s]).wait()

            start_gathers(0, 0)

            def pair_body(i2, _):
                for s in (0, 1):
                    c = 2 * i2 + s

                    @pl.when(c + 1 < n_chunks)
                    def _():
                        start_gathers(c + 1, 1 - s)

                    wait_gathers(s)
                    if first:
                        @pl.when(c >= 2)
                        def _():
                            wait_store(s)
                    else:
                        wait_store(s)

                    @plsc.parallel_loop(0, GC, unroll=8)
                    def _(r):
                        for j in range(D // L):
                            sl = pl.ds(j * L, L)
                            pv[s][r, sl] = xj[s][r, sl] * xd[s][r, sl]

                    base = pl.multiple_of(w_base + off(c), 8)
                    pltpu.async_copy(pv[s], p_hbm.at[pl.ds(base, GC)], ssem[s])
                return 0

            lax.fori_loop(0, n_chunks // 2, pair_body, 0)

        one_type(xu_hbm, xi_hbm, sui_hbm, dui_hbm, p_ui_hbm, True)
        one_type(xi_hbm, xu_hbm, siu_hbm, diu_hbm, p_iu_hbm, False)
        pltpu.make_async_copy(pv[0], p_iu_hbm.at[pl.ds(0, GC)], ssem[0]).wait()
        pltpu.make_async_copy(pv[1], p_iu_hbm.at[pl.ds(0, GC)], ssem[1]).wait()

    return gather_mul(x_user, x_item, src_ui, dst_ui, src_iu, dst_iu)


def _linear_lrelu_call(p_ui, p_iu, W_s, b_s):
    E = p_ui.shape[0]
    BLK = 2000

    def body(p1_ref, p2_ref, W_ref, b_ref, o1_ref, o2_ref):
        W = W_ref[...]
        b = b_ref[...]
        dn = (((1,), (1,)), ((), ()))
        z1 = lax.dot_general(p1_ref[...].astype(jnp.float32), W[0], dn,
                             preferred_element_type=jnp.float32) + b[0]
        z2 = lax.dot_general(p2_ref[...].astype(jnp.float32), W[1], dn,
                             preferred_element_type=jnp.float32) + b[1]
        o1_ref[...] = jnp.where(z1 >= 0, z1, 0.01 * z1)
        o2_ref[...] = jnp.where(z2 >= 0, z2, 0.01 * z2)

    return pl.pallas_call(
        body,
        grid=(E // BLK,),
        in_specs=[
            pl.BlockSpec((BLK, D), lambda i: (i, 0)),
            pl.BlockSpec((BLK, D), lambda i: (i, 0)),
            pl.BlockSpec((2, D, D), lambda i: (0, 0, 0)),
            pl.BlockSpec((2, D), lambda i: (0, 0)),
        ],
        out_specs=[
            pl.BlockSpec((BLK, D), lambda i: (i, 0)),
            pl.BlockSpec((BLK, D), lambda i: (i, 0)),
        ],
        out_shape=(jax.ShapeDtypeStruct((E, D), jnp.float32),
                   jax.ShapeDtypeStruct((E, D), jnp.float32)),
    )(p_ui, p_iu, W_s, b_s)


def _scatter_call(m_ui, dm_ui, dt_ui, m_iu, dm_iu, dt_iu, zeros_nd,
                  n_item, n_user):
    E = m_ui.shape[0]
    per_t = E // NS
    n_main = (per_t - SCT) // SCC
    n_rows = n_item
    stripe = 624
    stripe_last = n_rows - (NS - 1) * stripe
    K = 2

    mesh = plsc.VectorSubcoreMesh(core_axis_name="c", subcore_axis_name="s")

    @functools.partial(
        pl.kernel,
        mesh=mesh,
        out_type=(jax.ShapeDtypeStruct((n_item, D), jnp.float32),
                  jax.ShapeDtypeStruct((n_user, D), jnp.float32)),
        scratch_types=[
            pltpu.VMEM((n_main, SCC), jnp.int32),
            pltpu.VMEM((1, SCT), jnp.int32),
            pltpu.VMEM((SCC, D), jnp.float32),
            pltpu.VMEM((SCC, D), jnp.float32),
            pltpu.VMEM((SCT, D), jnp.float32),
            pltpu.VMEM_SHARED((10000, D), jnp.float32),
            pltpu.SemaphoreType.DMA,
            pltpu.SemaphoreType.DMA,
            pltpu.SemaphoreType.DMA,
            pltpu.SemaphoreType.DMA,
        ],
    )
    def scatter(mui_hbm, dmui_hbm, dtui_hbm, miu_hbm, dmiu_hbm, dtiu_hbm,
                z_hbm, oi_hbm, ou_hbm,
                d_all, d_tail, mr0, mr1, mtail, acc_sh,
                l0, l1, c0, c1):
        mrow = (mr0, mr1)
        lsem = (l0, l1)
        csem = (c0, c1)
        cid = lax.axis_index("c")
        tid = lax.axis_index("s")
        t_base = tid * per_t

        @pl.when(tid < NS - 1)
        def _():
            off = pl.multiple_of(tid * stripe, 8)
            pltpu.sync_copy(z_hbm.at[pl.ds(off, stripe)],
                            acc_sh.at[pl.ds(off, stripe)])

        @pl.when(tid == NS - 1)
        def _():
            off = (NS - 1) * stripe
            pltpu.sync_copy(z_hbm.at[pl.ds(off, stripe_last)],
                            acc_sh.at[pl.ds(off, stripe_last)])

        plsc.subcore_barrier()

        def one_type(m_hbm, dm_hbm, dt_hbm):
            pltpu.sync_copy(dm_hbm.at[tid], d_all)
            pltpu.sync_copy(dt_hbm.at[tid], d_tail)

            def start_load(c, s):
                base = pl.multiple_of(t_base + c * SCC, 8)
                pltpu.async_copy(m_hbm.at[pl.ds(base, SCC)], mrow[s], lsem[s])

            def wait_load(s):
                pltpu.make_async_copy(m_hbm.at[pl.ds(t_base, SCC)],
                                      mrow[s], lsem[s]).wait()

            def wait_scat(s):
                pltpu.make_async_copy(mrow[s], acc_sh.at[d_all.at[0]],
                                      csem[s]).wait()

            def step(c, s):
                wait_load(s)
                pltpu.async_copy(mrow[s], acc_sh.at[d_all.at[c]], csem[s],
                                 add=True)
                sprev = (s + K - 1) % K

                @pl.when(c >= 1)
                def _():
                    wait_scat(sprev)

                @pl.when(c + K - 1 < n_main)
                def _():
                    start_load(c + K - 1, sprev)

            for s in range(K - 1):
                start_load(s, s)

            n_loop = n_main - (n_main % K)

            def ring_body(i4, _):
                for s in range(K):
                    step(i4 * K + s, s)
                return 0

            lax.fori_loop(0, n_loop // K, ring_body, 0)
            for c in range(n_loop, n_main):
                step(c, c % K)
            wait_scat((n_main - 1) % K)
            tb = t_base + n_main * SCC
            pltpu.sync_copy(m_hbm.at[pl.ds(tb, SCT)], mtail)
            pltpu.sync_copy(mtail, acc_sh.at[d_tail.at[0]], add=True)

        @pl.when(cid == 0)
        def _():
            one_type(mui_hbm, dmui_hbm, dtui_hbm)

        @pl.when(cid == 1)
        def _():
            one_type(miu_hbm, dmiu_hbm, dtiu_hbm)

        plsc.subcore_barrier()

        def dump(o_hbm):
            @pl.when(tid < NS - 1)
            def _():
                off = pl.multiple_of(tid * stripe, 8)
                pltpu.sync_copy(acc_sh.at[pl.ds(off, stripe)],
                                o_hbm.at[pl.ds(off, stripe)])

            @pl.when(tid == NS - 1)
            def _():
                off = (NS - 1) * stripe
                pltpu.sync_copy(acc_sh.at[pl.ds(off, stripe_last)],
                                o_hbm.at[pl.ds(off, stripe_last)])

        @pl.when(cid == 0)
        def _():
            dump(oi_hbm)

        @pl.when(cid == 1)
        def _():
            dump(ou_hbm)

    return scatter(m_ui, dm_ui, dt_ui, m_iu, dm_iu, dt_iu, zeros_nd)


def _ln_relu_call(oi_raw, ou_raw, ln_g_item, ln_b_item, ln_g_user, ln_b_user):
    n = oi_raw.shape[0]
    BLKN = 2000

    def body(x1_ref, x2_ref, g1_ref, b1_ref, g2_ref, b2_ref, o1_ref, o2_ref):
        for x_ref, g_ref, b_ref, o_ref in (
            (x1_ref, g1_ref, b1_ref, o1_ref),
            (x2_ref, g2_ref, b2_ref, o2_ref),
        ):
            x = x_ref[...]
            mu = jnp.mean(x, axis=-1, keepdims=True)
            var = jnp.mean((x - mu) ** 2, axis=-1, keepdims=True)
            y = (x - mu) / jnp.sqrt(var + 1e-5) * g_ref[...] + b_ref[...]
            o_ref[...] = jnp.maximum(y, 0.0)

    vec_spec = pl.BlockSpec((1, D), lambda i: (0, 0))
    return pl.pallas_call(
        body,
        grid=(n // BLKN,),
        in_specs=[
            pl.BlockSpec((BLKN, D), lambda i: (i, 0)),
            pl.BlockSpec((BLKN, D), lambda i: (i, 0)),
            vec_spec, vec_spec, vec_spec, vec_spec,
        ],
        out_specs=[
            pl.BlockSpec((BLKN, D), lambda i: (i, 0)),
            pl.BlockSpec((BLKN, D), lambda i: (i, 0)),
        ],
        out_shape=(jax.ShapeDtypeStruct((n, D), jnp.float32),
                   jax.ShapeDtypeStruct((n, D), jnp.float32)),
    )(oi_raw, ou_raw, ln_g_item.reshape(1, D), ln_b_item.reshape(1, D),
      ln_g_user.reshape(1, D), ln_b_user.reshape(1, D))


def kernel(x_user, x_item, edge_index_user_item, edge_index_item_user,
           W_ui, b_ui, W_iu, b_iu,
           ln_g_user, ln_b_user, ln_g_item, ln_b_item):
    n_user = x_user.shape[0]
    n_item = x_item.shape[0]
    E = edge_index_user_item.shape[1]

    src_ui = edge_index_user_item[0].astype(jnp.int32)
    dst_ui = edge_index_user_item[1].astype(jnp.int32)
    src_iu = edge_index_item_user[0].astype(jnp.int32)
    dst_iu = edge_index_item_user[1].astype(jnp.int32)

    p_ui, p_iu = _gather_mul_call(x_user, x_item, src_ui, dst_ui,
                                  src_iu, dst_iu)

    W_s = jnp.stack([W_ui, W_iu])
    b_s = jnp.stack([b_ui, b_iu])
    m_ui, m_iu = _linear_lrelu_call(p_ui, p_iu, W_s, b_s)

    per_t = E // NS
    dd_ui = dst_ui.reshape(NS, per_t)
    dd_iu = dst_iu.reshape(NS, per_t)
    n_main = (per_t - SCT) // SCC
    dm_ui = dd_ui[:, :n_main * SCC].reshape(NS, n_main, SCC)
    dt_ui = dd_ui[:, n_main * SCC:].reshape(NS, 1, SCT)
    dm_iu = dd_iu[:, :n_main * SCC].reshape(NS, n_main, SCC)
    dt_iu = dd_iu[:, n_main * SCC:].reshape(NS, 1, SCT)

    zeros_nd = jnp.zeros((n_item, D), jnp.float32)
    oi_raw, ou_raw = _scatter_call(m_ui, dm_ui, dt_ui, m_iu, dm_iu, dt_iu,
                                   zeros_nd, n_item, n_user)

    out_item, out_user = _ln_relu_call(oi_raw, ou_raw,
                                       ln_g_item, ln_b_item,
                                       ln_g_user, ln_b_user)
    return (out_user, out_item)


# speedup vs baseline: 1.2225x; 1.0613x over previous
import functools

import jax
import jax.numpy as jnp
from jax import lax
from jax.experimental import pallas as pl
from jax.experimental.pallas import tpu as pltpu
from jax.experimental.pallas import tpu_sc as plsc

D = 128
L = 16
NC = 2
NS = 16
NW = NC * NS

GC = 128
SCC = 128
SCT = 8


def _gather_mul_call(x_src, x_dst, src_idx, dst_idx):
    E = src_idx.shape[0]
    per_w = E // NW
    n_chunks = -(-per_w // GC)
    last_base = per_w - GC
    assert n_chunks % 2 == 0

    mesh = plsc.VectorSubcoreMesh(core_axis_name="c", subcore_axis_name="s")

    @functools.partial(
        pl.kernel,
        mesh=mesh,
        out_type=jax.ShapeDtypeStruct((E, D), jnp.float32),
        scratch_types=[
            pltpu.VMEM((per_w,), jnp.int32),
            pltpu.VMEM((per_w,), jnp.int32),
            pltpu.VMEM((GC, D), jnp.float32),
            pltpu.VMEM((GC, D), jnp.float32),
            pltpu.VMEM((GC, D), jnp.float32),
            pltpu.VMEM((GC, D), jnp.float32),
            pltpu.VMEM((GC, D), jnp.float32),
            pltpu.VMEM((GC, D), jnp.float32),
            pltpu.SemaphoreType.DMA,
            pltpu.SemaphoreType.DMA,
            pltpu.SemaphoreType.DMA,
            pltpu.SemaphoreType.DMA,
        ],
    )
    def gather_mul(xs_hbm, xdst_hbm, s_hbm, d_hbm, p_hbm,
                   si_all, di_all, xj0, xj1, xd0, xd1, p0, p1,
                   gs0, gs1, ss0, ss1):
        xj = (xj0, xj1)
        xd = (xd0, xd1)
        pv = (p0, p1)
        gsem = (gs0, gs1)
        ssem = (ss0, ss1)
        wid = lax.axis_index("s") * NC + lax.axis_index("c")
        w_base = wid * per_w

        pltpu.sync_copy(s_hbm.at[pl.ds(w_base, per_w)], si_all)
        pltpu.sync_copy(d_hbm.at[pl.ds(w_base, per_w)], di_all)

        def off(c):
            return pl.multiple_of(jnp.minimum(c * GC, last_base), 8)

        def start_gathers(c, s):
            o = off(c)
            pltpu.async_copy(xs_hbm.at[si_all.at[pl.ds(o, GC)]], xj[s], gsem[s])
            pltpu.async_copy(xdst_hbm.at[di_all.at[pl.ds(o, GC)]], xd[s], gsem[s])

        def wait_gathers(s):
            z = pl.ds(0, GC)
            pltpu.make_async_copy(xs_hbm.at[si_all.at[z]], xj[s], gsem[s]).wait()
            pltpu.make_async_copy(xdst_hbm.at[di_all.at[z]], xd[s], gsem[s]).wait()

        def wait_store(s):
            pltpu.make_async_copy(pv[s], p_hbm.at[pl.ds(0, GC)], ssem[s]).wait()

        start_gathers(0, 0)

        def pair_body(i2, _):
            for s in (0, 1):
                c = 2 * i2 + s

                @pl.when(c + 1 < n_chunks)
                def _():
                    start_gathers(c + 1, 1 - s)

                wait_gathers(s)

                @pl.when(c >= 2)
                def _():
                    wait_store(s)

                @plsc.parallel_loop(0, GC, unroll=8)
                def _(r):
                    for j in range(D // L):
                        sl = pl.ds(j * L, L)
                        pv[s][r, sl] = xj[s][r, sl] * xd[s][r, sl]

                base = pl.multiple_of(w_base + off(c), 8)
                pltpu.async_copy(pv[s], p_hbm.at[pl.ds(base, GC)], ssem[s])
            return 0

        lax.fori_loop(0, n_chunks // 2, pair_body, 0)
        wait_store(0)
        wait_store(1)

    return gather_mul(x_src, x_dst, src_idx, dst_idx)


def _linear_lrelu_call(p, W, b):
    E = p.shape[0]
    BLK = 2000

    def body(p_ref, W_ref, b_ref, o_ref):
        dn = (((1,), (1,)), ((), ()))
        z = lax.dot_general(p_ref[...], W_ref[...], dn,
                            preferred_element_type=jnp.float32) + b_ref[...]
        o_ref[...] = jnp.where(z >= 0, z, 0.01 * z)

    return pl.pallas_call(
        body,
        grid=(E // BLK,),
        in_specs=[
            pl.BlockSpec((BLK, D), lambda i: (i, 0)),
            pl.BlockSpec((D, D), lambda i: (0, 0)),
            pl.BlockSpec((1, D), lambda i: (0, 0)),
        ],
        out_specs=pl.BlockSpec((BLK, D), lambda i: (i, 0)),
        out_shape=jax.ShapeDtypeStruct((E, D), jnp.float32),
    )(p, W, b.reshape(1, D))


def _scatter_call(m, dm, dt, zeros_nd, n_rows):
    E = m.shape[0]
    per_t = E // NW
    n_main = (per_t - SCT) // SCC
    stripe = 624
    stripe_last = n_rows - (NS - 1) * stripe
    K = 2

    mesh = plsc.VectorSubcoreMesh(core_axis_name="c", subcore_axis_name="s")

    @functools.partial(
        pl.kernel,
        mesh=mesh,
        out_type=jax.ShapeDtypeStruct((NC, n_rows, D), jnp.float32),
        scratch_types=[
            pltpu.VMEM((n_main, SCC), jnp.int32),
            pltpu.VMEM((1, SCT), jnp.int32),
            pltpu.VMEM((SCC, D), jnp.float32),
            pltpu.VMEM((SCC, D), jnp.float32),
            pltpu.VMEM((SCT, D), jnp.float32),
            pltpu.VMEM_SHARED((10000, D), jnp.float32),
            pltpu.SemaphoreType.DMA,
            pltpu.SemaphoreType.DMA,
            pltpu.SemaphoreType.DMA,
            pltpu.SemaphoreType.DMA,
        ],
    )
    def scatter(m_hbm, dm_hbm, dt_hbm, z_hbm, o_hbm,
                d_all, d_tail, mr0, mr1, mtail, acc_sh,
                l0, l1, c0, c1):
        mrow = (mr0, mr1)
        lsem = (l0, l1)
        csem = (c0, c1)
        cid = lax.axis_index("c")
        tid = lax.axis_index("s")
        wid = tid * NC + cid
        t_base = wid * per_t

        @pl.when(tid < NS - 1)
        def _():
            off = pl.multiple_of(tid * stripe, 8)
            pltpu.sync_copy(z_hbm.at[pl.ds(off, stripe)],
                            acc_sh.at[pl.ds(off, stripe)])

        @pl.when(tid == NS - 1)
        def _():
            off = (NS - 1) * stripe
            pltpu.sync_copy(z_hbm.at[pl.ds(off, stripe_last)],
                            acc_sh.at[pl.ds(off, stripe_last)])

        pltpu.sync_copy(dm_hbm.at[wid], d_all)
        pltpu.sync_copy(dt_hbm.at[wid], d_tail)
        plsc.subcore_barrier()

        def start_load(c, s):
            base = pl.multiple_of(t_base + c * SCC, 8)
            pltpu.async_copy(m_hbm.at[pl.ds(base, SCC)], mrow[s], lsem[s])

        def wait_load(s):
            pltpu.make_async_copy(m_hbm.at[pl.ds(t_base, SCC)],
                                  mrow[s], lsem[s]).wait()

        def wait_scat(s):
            pltpu.make_async_copy(mrow[s], acc_sh.at[d_all.at[0]],
                                  csem[s]).wait()

        def step(c, s):
            wait_load(s)
            pltpu.async_copy(mrow[s], acc_sh.at[d_all.at[c]], csem[s],
                             add=True)
            sprev = (s + K - 1) % K

            @pl.when(c >= 1)
            def _():
                wait_scat(sprev)

            @pl.when(c + K - 1 < n_main)
            def _():
                start_load(c + K - 1, sprev)

        for s in range(K - 1):
            start_load(s, s)

        n_loop = n_main - (n_main % K)

        def ring_body(i4, _):
            for s in range(K):
                step(i4 * K + s, s)
            return 0

        lax.fori_loop(0, n_loop // K, ring_body, 0)
        for c in range(n_loop, n_main):
            step(c, c % K)
        wait_scat((n_main - 1) % K)
        tb = t_base + n_main * SCC
        pltpu.sync_copy(m_hbm.at[pl.ds(tb, SCT)], mtail)
        pltpu.sync_copy(mtail, acc_sh.at[d_tail.at[0]], add=True)

        plsc.subcore_barrier()

        @pl.when(tid < NS - 1)
        def _():
            off = pl.multiple_of(tid * stripe, 8)
            pltpu.sync_copy(acc_sh.at[pl.ds(off, stripe)],
                            o_hbm.at[cid, pl.ds(off, stripe)])

        @pl.when(tid == NS - 1)
        def _():
            off = (NS - 1) * stripe
            pltpu.sync_copy(acc_sh.at[pl.ds(off, stripe_last)],
                            o_hbm.at[cid, pl.ds(off, stripe_last)])

    return scatter(m, dm, dt, zeros_nd)


def _ln_relu_call(oi_parts, ou_parts, ln_g_item, ln_b_item, ln_g_user, ln_b_user):
    n = oi_parts.shape[1]
    BLKN = 2000

    def body(x1_ref, x2_ref, g1_ref, b1_ref, g2_ref, b2_ref, o1_ref, o2_ref):
        for x_ref, g_ref, b_ref, o_ref in (
            (x1_ref, g1_ref, b1_ref, o1_ref),
            (x2_ref, g2_ref, b2_ref, o2_ref),
        ):
            x = x_ref[0] + x_ref[1]
            mu = jnp.mean(x, axis=-1, keepdims=True)
            var = jnp.mean((x - mu) ** 2, axis=-1, keepdims=True)
            y = (x - mu) / jnp.sqrt(var + 1e-5) * g_ref[...] + b_ref[...]
            o_ref[...] = jnp.maximum(y, 0.0)

    part_spec = pl.BlockSpec((NC, BLKN, D), lambda i: (0, i, 0))
    vec_spec = pl.BlockSpec((1, D), lambda i: (0, 0))
    return pl.pallas_call(
        body,
        grid=(n // BLKN,),
        in_specs=[part_spec, part_spec, vec_spec, vec_spec, vec_spec, vec_spec],
        out_specs=[
            pl.BlockSpec((BLKN, D), lambda i: (i, 0)),
            pl.BlockSpec((BLKN, D), lambda i: (i, 0)),
        ],
        out_shape=(jax.ShapeDtypeStruct((n, D), jnp.float32),
                   jax.ShapeDtypeStruct((n, D), jnp.float32)),
    )(oi_parts, ou_parts, ln_g_item.reshape(1, D), ln_b_item.reshape(1, D),
      ln_g_user.reshape(1, D), ln_b_user.reshape(1, D))


def kernel(x_user, x_item, edge_index_user_item, edge_index_item_user,
           W_ui, b_ui, W_iu, b_iu,
           ln_g_user, ln_b_user, ln_g_item, ln_b_item):
    n_user = x_user.shape[0]
    n_item = x_item.shape[0]
    E = edge_index_user_item.shape[1]

    src_ui = edge_index_user_item[0].astype(jnp.int32)
    dst_ui = edge_index_user_item[1].astype(jnp.int32)
    src_iu = edge_index_item_user[0].astype(jnp.int32)
    dst_iu = edge_index_item_user[1].astype(jnp.int32)

    per_t = E // NW
    n_main = (per_t - SCT) // SCC

    def dst_split(dst):
        dd = dst.reshape(NW, per_t)
        dm = dd[:, :n_main * SCC].reshape(NW, n_main, SCC)
        dt = dd[:, n_main * SCC:].reshape(NW, 1, SCT)
        return dm, dt

    dm_ui, dt_ui = dst_split(dst_ui)
    dm_iu, dt_iu = dst_split(dst_iu)
    zeros_nd = jnp.zeros((n_item, D), jnp.float32)

    p_ui = _gather_mul_call(x_user, x_item, src_ui, dst_ui)
    m_ui = _linear_lrelu_call(p_ui, W_ui, b_ui)
    p_iu = _gather_mul_call(x_item, x_user, src_iu, dst_iu)
    m_iu = _linear_lrelu_call(p_iu, W_iu, b_iu)

    oi_parts = _scatter_call(m_ui, dm_ui, dt_ui, zeros_nd, n_item)
    ou_parts = _scatter_call(m_iu, dm_iu, dt_iu, zeros_nd, n_user)

    out_item, out_user = _ln_relu_call(oi_parts, ou_parts,
                                       ln_g_item, ln_b_item,
                                       ln_g_user, ln_b_user)
    return (out_user, out_item)


# speedup vs baseline: 1.2363x; 1.0113x over previous
import functools

import jax
import jax.numpy as jnp
from jax import lax
from jax.experimental import pallas as pl
from jax.experimental.pallas import tpu as pltpu
from jax.experimental.pallas import tpu_sc as plsc

D = 128
L = 16
NC = 2
NS = 16
NW = NC * NS

GC = 128
SCC = 128
SCT = 8


def _gather_mul_call(x_src, x_dst, src_idx, dst_idx):
    E = src_idx.shape[0]
    per_w = E // NW
    n_chunks = -(-per_w // GC)
    last_base = per_w - GC
    assert n_chunks % 2 == 0

    mesh = plsc.VectorSubcoreMesh(core_axis_name="c", subcore_axis_name="s")

    @functools.partial(
        pl.kernel,
        mesh=mesh,
        out_type=jax.ShapeDtypeStruct((E, D), jnp.float32),
        scratch_types=[
            pltpu.VMEM((per_w,), jnp.int32),
            pltpu.VMEM((per_w,), jnp.int32),
            pltpu.VMEM((GC, D), jnp.float32),
            pltpu.VMEM((GC, D), jnp.float32),
            pltpu.VMEM((GC, D), jnp.float32),
            pltpu.VMEM((GC, D), jnp.float32),
            pltpu.VMEM((GC, D), jnp.float32),
            pltpu.VMEM((GC, D), jnp.float32),
            pltpu.SemaphoreType.DMA,
            pltpu.SemaphoreType.DMA,
            pltpu.SemaphoreType.DMA,
            pltpu.SemaphoreType.DMA,
        ],
    )
    def gather_mul(xs_hbm, xdst_hbm, s_hbm, d_hbm, p_hbm,
                   si_all, di_all, xj0, xj1, xd0, xd1, p0, p1,
                   gs0, gs1, ss0, ss1):
        xj = (xj0, xj1)
        xd = (xd0, xd1)
        pv = (p0, p1)
        gsem = (gs0, gs1)
        ssem = (ss0, ss1)
        wid = lax.axis_index("s") * NC + lax.axis_index("c")
        w_base = wid * per_w

        pltpu.sync_copy(s_hbm.at[pl.ds(w_base, per_w)], si_all)
        pltpu.sync_copy(d_hbm.at[pl.ds(w_base, per_w)], di_all)

        def off(c):
            return pl.multiple_of(jnp.minimum(c * GC, last_base), 8)

        def start_gathers(c, s):
            o = off(c)
            pltpu.async_copy(xs_hbm.at[si_all.at[pl.ds(o, GC)]], xj[s], gsem[s])
            pltpu.async_copy(xdst_hbm.at[di_all.at[pl.ds(o, GC)]], xd[s], gsem[s])

        def wait_gathers(s):
            z = pl.ds(0, GC)
            pltpu.make_async_copy(xs_hbm.at[si_all.at[z]], xj[s], gsem[s]).wait()
            pltpu.make_async_copy(xdst_hbm.at[di_all.at[z]], xd[s], gsem[s]).wait()

        def wait_store(s):
            pltpu.make_async_copy(pv[s], p_hbm.at[pl.ds(0, GC)], ssem[s]).wait()

        start_gathers(0, 0)

        def pair_body(i2, _):
            for s in (0, 1):
                c = 2 * i2 + s

                @pl.when(c + 1 < n_chunks)
                def _():
                    start_gathers(c + 1, 1 - s)

                wait_gathers(s)

                @pl.when(c >= 2)
                def _():
                    wait_store(s)

                @plsc.parallel_loop(0, GC, unroll=8)
                def _(r):
                    for j in range(D // L):
                        sl = pl.ds(j * L, L)
                        pv[s][r, sl] = xj[s][r, sl] * xd[s][r, sl]

                base = pl.multiple_of(w_base + off(c), 8)
                pltpu.async_copy(pv[s], p_hbm.at[pl.ds(base, GC)], ssem[s])
            return 0

        lax.fori_loop(0, n_chunks // 2, pair_body, 0)
        wait_store(0)
        wait_store(1)

    return gather_mul(x_src, x_dst, src_idx, dst_idx)


def _linear_lrelu_call(p, W, b):
    E = p.shape[0]
    BLK = 2000

    def body(p_ref, W_ref, b_ref, o_ref):
        dn = (((1,), (1,)), ((), ()))
        z = lax.dot_general(p_ref[...], W_ref[...], dn,
                            preferred_element_type=jnp.float32) + b_ref[...]
        o_ref[...] = jnp.where(z >= 0, z, 0.01 * z)

    return pl.pallas_call(
        body,
        grid=(E // BLK,),
        in_specs=[
            pl.BlockSpec((BLK, D), lambda i: (i, 0)),
            pl.BlockSpec((D, D), lambda i: (0, 0)),
            pl.BlockSpec((1, D), lambda i: (0, 0)),
        ],
        out_specs=pl.BlockSpec((BLK, D), lambda i: (i, 0)),
        out_shape=jax.ShapeDtypeStruct((E, D), jnp.float32),
    )(p, W, b.reshape(1, D))


def _scatter_call(m, dm, dt, zeros_nd, n_rows):
    E = m.shape[0]
    per_t = E // NW
    n_main = (per_t - SCT) // SCC
    stripe = 624
    stripe_last = n_rows - (NS - 1) * stripe
    K = 2

    mesh = plsc.VectorSubcoreMesh(core_axis_name="c", subcore_axis_name="s")

    @functools.partial(
        pl.kernel,
        mesh=mesh,
        out_type=jax.ShapeDtypeStruct((NC, n_rows, D), jnp.float32),
        scratch_types=[
            pltpu.VMEM((n_main, SCC), jnp.int32),
            pltpu.VMEM((1, SCT), jnp.int32),
            pltpu.VMEM((SCC, D), jnp.float32),
            pltpu.VMEM((SCC, D), jnp.float32),
            pltpu.VMEM((SCT, D), jnp.float32),
            pltpu.VMEM_SHARED((10000, D), jnp.float32),
            pltpu.SemaphoreType.DMA,
            pltpu.SemaphoreType.DMA,
            pltpu.SemaphoreType.DMA,
            pltpu.SemaphoreType.DMA,
        ],
    )
    def scatter(m_hbm, dm_hbm, dt_hbm, z_hbm, o_hbm,
                d_all, d_tail, mr0, mr1, mtail, acc_sh,
                l0, l1, c0, c1):
        mrow = (mr0, mr1)
        lsem = (l0, l1)
        csem = (c0, c1)
        cid = lax.axis_index("c")
        tid = lax.axis_index("s")
        wid = tid * NC + cid
        t_base = wid * per_t

        @pl.when(tid < NS - 1)
        def _():
            off = pl.multiple_of(tid * stripe, 8)
            pltpu.sync_copy(z_hbm.at[pl.ds(off, stripe)],
                            acc_sh.at[pl.ds(off, stripe)])

        @pl.when(tid == NS - 1)
        def _():
            off = (NS - 1) * stripe
            pltpu.sync_copy(z_hbm.at[pl.ds(off, stripe_last)],
                            acc_sh.at[pl.ds(off, stripe_last)])

        pltpu.sync_copy(dm_hbm.at[wid], d_all)
        pltpu.sync_copy(dt_hbm.at[wid], d_tail)
        plsc.subcore_barrier()

        def start_load(c, s):
            base = pl.multiple_of(t_base + c * SCC, 8)
            pltpu.async_copy(m_hbm.at[pl.ds(base, SCC)], mrow[s], lsem[s])

        def wait_load(s):
            pltpu.make_async_copy(m_hbm.at[pl.ds(t_base, SCC)],
                                  mrow[s], lsem[s]).wait()

        def wait_scat(s):
            pltpu.make_async_copy(mrow[s], acc_sh.at[d_all.at[0]],
                                  csem[s]).wait()

        def step(c, s):
            wait_load(s)
            pltpu.async_copy(mrow[s], acc_sh.at[d_all.at[c]], csem[s],
                             add=True)
            sprev = (s + K - 1) % K

            @pl.when(c >= 1)
            def _():
                wait_scat(sprev)

            @pl.when(c + K - 1 < n_main)
            def _():
                start_load(c + K - 1, sprev)

        for s in range(K - 1):
            start_load(s, s)

        n_loop = n_main - (n_main % K)

        def ring_body(i4, _):
            for s in range(K):
                step(i4 * K + s, s)
            return 0

        lax.fori_loop(0, n_loop // K, ring_body, 0)
        for c in range(n_loop, n_main):
            step(c, c % K)
        wait_scat((n_main - 1) % K)
        tb = t_base + n_main * SCC
        pltpu.sync_copy(m_hbm.at[pl.ds(tb, SCT)], mtail)
        pltpu.sync_copy(mtail, acc_sh.at[d_tail.at[0]], add=True)

        plsc.subcore_barrier()

        @pl.when(tid < NS - 1)
        def _():
            off = pl.multiple_of(tid * stripe, 8)
            pltpu.sync_copy(acc_sh.at[pl.ds(off, stripe)],
                            o_hbm.at[cid, pl.ds(off, stripe)])

        @pl.when(tid == NS - 1)
        def _():
            off = (NS - 1) * stripe
            pltpu.sync_copy(acc_sh.at[pl.ds(off, stripe_last)],
                            o_hbm.at[cid, pl.ds(off, stripe_last)])

    return scatter(m, dm, dt, zeros_nd)


def _ln_relu_call(parts, ln_g, ln_b):
    n = parts.shape[1]
    BLKN = 2000

    def body(x_ref, g_ref, b_ref, o_ref):
        x = x_ref[0] + x_ref[1]
        mu = jnp.mean(x, axis=-1, keepdims=True)
        var = jnp.mean((x - mu) ** 2, axis=-1, keepdims=True)
        y = (x - mu) / jnp.sqrt(var + 1e-5) * g_ref[...] + b_ref[...]
        o_ref[...] = jnp.maximum(y, 0.0)

    part_spec = pl.BlockSpec((NC, BLKN, D), lambda i: (0, i, 0))
    vec_spec = pl.BlockSpec((1, D), lambda i: (0, 0))
    return pl.pallas_call(
        body,
        grid=(n // BLKN,),
        in_specs=[part_spec, vec_spec, vec_spec],
        out_specs=pl.BlockSpec((BLKN, D), lambda i: (i, 0)),
        out_shape=jax.ShapeDtypeStruct((n, D), jnp.float32),
    )(parts, ln_g.reshape(1, D), ln_b.reshape(1, D))


def kernel(x_user, x_item, edge_index_user_item, edge_index_item_user,
           W_ui, b_ui, W_iu, b_iu,
           ln_g_user, ln_b_user, ln_g_item, ln_b_item):
    n_user = x_user.shape[0]
    n_item = x_item.shape[0]
    E = edge_index_user_item.shape[1]

    src_ui = edge_index_user_item[0].astype(jnp.int32)
    dst_ui = edge_index_user_item[1].astype(jnp.int32)
    src_iu = edge_index_item_user[0].astype(jnp.int32)
    dst_iu = edge_index_item_user[1].astype(jnp.int32)

    per_t = E // NW
    n_main = (per_t - SCT) // SCC

    def dst_split(dst):
        dd = dst.reshape(NW, per_t)
        dm = dd[:, :n_main * SCC].reshape(NW, n_main, SCC)
        dt = dd[:, n_main * SCC:].reshape(NW, 1, SCT)
        return dm, dt

    dm_ui, dt_ui = dst_split(dst_ui)
    dm_iu, dt_iu = dst_split(dst_iu)
    zeros_nd = jnp.zeros((n_item, D), jnp.float32)

    p_ui = _gather_mul_call(x_user, x_item, src_ui, dst_ui)
    m_ui = _linear_lrelu_call(p_ui, W_ui, b_ui)
    p_iu = _gather_mul_call(x_item, x_user, src_iu, dst_iu)

    oi_parts = _scatter_call(m_ui, dm_ui, dt_ui, zeros_nd, n_item)
    m_iu = _linear_lrelu_call(p_iu, W_iu, b_iu)
    out_item = _ln_relu_call(oi_parts, ln_g_item, ln_b_item)

    ou_parts = _scatter_call(m_iu, dm_iu, dt_iu, zeros_nd, n_user)
    out_user = _ln_relu_call(ou_parts, ln_g_user, ln_b_user)
    return (out_user, out_item)
